# baseline (speedup 1.0000x reference)
import jax
import jax.numpy as jnp
from jax.experimental import pallas as pl
from jax.experimental.pallas import tpu as pltpu

_N = 1024 * 1024
_INV_T = 10.0
_CHUNKS = 8
_CH = _N // _CHUNKS


def _body(w_hbm, z_hbm, o_hbm, w_v, z_v, o_v, w_sem, z_sem, o_sem):
    def start_in(i):
        slot = jax.lax.rem(i, 2)
        pltpu.make_async_copy(w_hbm.at[pl.ds(i * _CH, _CH)], w_v.at[slot],
                              w_sem.at[slot]).start()
        pltpu.make_async_copy(z_hbm.at[pl.ds(i * _CH, _CH)], z_v.at[slot],
                              z_sem.at[slot]).start()

    start_in(0)

    def step(i, carry):
        slot = jax.lax.rem(i, 2)

        @pl.when(i + 1 < _CHUNKS)
        def _():
            start_in(i + 1)

        pltpu.make_async_copy(w_hbm.at[pl.ds(i * _CH, _CH)], w_v.at[slot],
                              w_sem.at[slot]).wait()
        pltpu.make_async_copy(z_hbm.at[pl.ds(i * _CH, _CH)], z_v.at[slot],
                              z_sem.at[slot]).wait()

        @pl.when(i >= 2)
        def _():
            pltpu.make_async_copy(o_v.at[slot],
                                  o_hbm.at[pl.ds((i - 2) * _CH, _CH)],
                                  o_sem.at[slot]).wait()

        x = (w_v[slot] - z_v[slot]) * _INV_T
        o_v[slot] = jax.nn.sigmoid(x)
        pltpu.make_async_copy(o_v.at[slot], o_hbm.at[pl.ds(i * _CH, _CH)],
                              o_sem.at[slot]).start()
        return carry

    jax.lax.fori_loop(0, _CHUNKS, step, 0)
    for tail in (_CHUNKS - 2, _CHUNKS - 1):
        slot = tail % 2
        pltpu.make_async_copy(o_v.at[slot], o_hbm.at[pl.ds(tail * _CH, _CH)],
                              o_sem.at[slot]).wait()


def kernel(weights, noises):
    return pl.pallas_call(
        _body,
        in_specs=[
            pl.BlockSpec(memory_space=pl.ANY),
            pl.BlockSpec(memory_space=pl.ANY),
        ],
        out_specs=pl.BlockSpec(memory_space=pl.ANY),
        out_shape=jax.ShapeDtypeStruct((_N,), jnp.float32),
        scratch_shapes=[
            pltpu.VMEM((2, _CH), jnp.float32),
            pltpu.VMEM((2, _CH), jnp.float32),
            pltpu.VMEM((2, _CH), jnp.float32),
            pltpu.SemaphoreType.DMA((2,)),
            pltpu.SemaphoreType.DMA((2,)),
            pltpu.SemaphoreType.DMA((2,)),
        ],
    )(weights, noises)


# speedup vs baseline: 1.2736x; 1.2736x over previous
import jax
import jax.numpy as jnp
from jax.experimental import pallas as pl
from jax.experimental.pallas import tpu as pltpu

_N = 1024 * 1024
_INV_T = 10.0
_CHUNKS = 4
_CH = _N // _CHUNKS


def _body(w_hbm, z_hbm, o_hbm, w_v, z_v, o_v, w_sem, z_sem, o_sem):
    def start_in(i):
        slot = jax.lax.rem(i, 2)
        pltpu.make_async_copy(w_hbm.at[pl.ds(i * _CH, _CH)], w_v.at[slot],
                              w_sem.at[slot]).start()
        pltpu.make_async_copy(z_hbm.at[pl.ds(i * _CH, _CH)], z_v.at[slot],
                              z_sem.at[slot]).start()

    start_in(0)

    def step(i, carry):
        slot = jax.lax.rem(i, 2)

        @pl.when(i + 1 < _CHUNKS)
        def _():
            start_in(i + 1)

        pltpu.make_async_copy(w_hbm.at[pl.ds(i * _CH, _CH)], w_v.at[slot],
                              w_sem.at[slot]).wait()
        pltpu.make_async_copy(z_hbm.at[pl.ds(i * _CH, _CH)], z_v.at[slot],
                              z_sem.at[slot]).wait()

        @pl.when(i >= 2)
        def _():
            pltpu.make_async_copy(o_v.at[slot],
                                  o_hbm.at[pl.ds((i - 2) * _CH, _CH)],
                                  o_sem.at[slot]).wait()

        x = (w_v[slot] - z_v[slot]) * _INV_T
        o_v[slot] = jax.nn.sigmoid(x)
        pltpu.make_async_copy(o_v.at[slot], o_hbm.at[pl.ds(i * _CH, _CH)],
                              o_sem.at[slot]).start()
        return carry

    jax.lax.fori_loop(0, _CHUNKS, step, 0)
    for tail in (_CHUNKS - 2, _CHUNKS - 1):
        slot = tail % 2
        pltpu.make_async_copy(o_v.at[slot], o_hbm.at[pl.ds(tail * _CH, _CH)],
                              o_sem.at[slot]).wait()


def kernel(weights, noises):
    return pl.pallas_call(
        _body,
        in_specs=[
            pl.BlockSpec(memory_space=pl.ANY),
            pl.BlockSpec(memory_space=pl.ANY),
        ],
        out_specs=pl.BlockSpec(memory_space=pl.ANY),
        out_shape=jax.ShapeDtypeStruct((_N,), jnp.float32),
        scratch_shapes=[
            pltpu.VMEM((2, _CH), jnp.float32),
            pltpu.VMEM((2, _CH), jnp.float32),
            pltpu.VMEM((2, _CH), jnp.float32),
            pltpu.SemaphoreType.DMA((2,)),
            pltpu.SemaphoreType.DMA((2,)),
            pltpu.SemaphoreType.DMA((2,)),
        ],
    )(weights, noises)


# speedup vs baseline: 1.4898x; 1.1698x over previous
import jax
import jax.numpy as jnp
from jax.experimental import pallas as pl

_N = 1024 * 1024
_INV_T = 10.0
_GRID = 2


def _body(w_ref, z_ref, o_ref):
    x = (w_ref[...] - z_ref[...]) * _INV_T
    o_ref[...] = jax.nn.sigmoid(x)


def kernel(weights, noises):
    blk = _N // _GRID
    out = pl.pallas_call(
        _body,
        grid=(_GRID,),
        in_specs=[
            pl.BlockSpec((blk,), lambda i: (i,)),
            pl.BlockSpec((blk,), lambda i: (i,)),
        ],
        out_specs=pl.BlockSpec((blk,), lambda i: (i,)),
        out_shape=jax.ShapeDtypeStruct((_N,), jnp.float32),
    )(weights, noises)
    return out
